# baseline (speedup 1.0000x reference)
import jax
import jax.numpy as jnp
from jax.experimental import pallas as pl
from jax.experimental.pallas import tpu as pltpu

B = 16384
N = 1000
MAIN = 896
TAIL = 128
BLOCK_B = 1024

_NEG_INF = float("-inf")


def _body(xm_ref, gm_ref, xt_ref, gt_ref, out_ref):
    zm = xm_ref[...] + gm_ref[...]
    zt = xt_ref[...] + gt_ref[...]
    m = jnp.maximum(jnp.max(zm, axis=1, keepdims=True),
                    jnp.max(zt, axis=1, keepdims=True))
    ssum = (jnp.sum(jnp.exp(zm - m), axis=1, keepdims=True)
            + jnp.sum(jnp.exp(zt - m), axis=1, keepdims=True))
    s = 1.0 / ssum
    val = (1.0 - s) + s

    cm = jax.lax.broadcasted_iota(jnp.int32, zm.shape, 1)
    ct = jax.lax.broadcasted_iota(jnp.int32, zt.shape, 1) + MAIN
    idx = jnp.minimum(
        jnp.min(jnp.where(zm == m, cm, N), axis=1, keepdims=True),
        jnp.min(jnp.where(zt == m, ct, N), axis=1, keepdims=True))

    cols = jax.lax.broadcasted_iota(jnp.int32, (BLOCK_B, N), 1)
    out = jnp.where(cols == idx, val, 0.0)

    rows = jax.lax.broadcasted_iota(jnp.int32, (BLOCK_B, N), 0) + pl.program_id(0) * BLOCK_B
    out_ref[...] = jnp.where((rows == 0) & (cols == 1), 1.0, out)


def kernel(x, gumbels):
    xt = jnp.pad(x[:, MAIN:], ((0, 0), (0, MAIN + TAIL - N)), constant_values=_NEG_INF)
    gt = jnp.pad(gumbels[:, MAIN:], ((0, 0), (0, MAIN + TAIL - N)), constant_values=0.0)
    return pl.pallas_call(
        _body,
        grid=(B // BLOCK_B,),
        in_specs=[
            pl.BlockSpec((BLOCK_B, MAIN), lambda i: (i, 0)),
            pl.BlockSpec((BLOCK_B, MAIN), lambda i: (i, 0)),
            pl.BlockSpec((BLOCK_B, TAIL), lambda i: (i, 0)),
            pl.BlockSpec((BLOCK_B, TAIL), lambda i: (i, 0)),
        ],
        out_specs=pl.BlockSpec((BLOCK_B, N), lambda i: (i, 0)),
        out_shape=jax.ShapeDtypeStruct((B, N), jnp.float32),
        compiler_params=pltpu.CompilerParams(
            dimension_semantics=("parallel",),
        ),
    )(x, gumbels, xt, gt)


# speedup vs baseline: 1.0687x; 1.0687x over previous
import jax
import jax.numpy as jnp
from jax.experimental import pallas as pl
from jax.experimental.pallas import tpu as pltpu

B = 16384
N = 1000
BLOCK_B = 2048


def _fused_body(x_ref, g_ref, out_ref):
    z = x_ref[...] + g_ref[...]
    m = jnp.max(z, axis=1, keepdims=True)
    ssum = jnp.sum(jnp.exp(z - m), axis=1, keepdims=True)
    s = 1.0 / ssum
    val = (1.0 - s) + s

    cols = jax.lax.broadcasted_iota(jnp.int32, z.shape, 1)
    idx = jnp.min(jnp.where(z == m, cols, N), axis=1, keepdims=True)
    out = jnp.where(cols == idx, val, 0.0)

    rows = jax.lax.broadcasted_iota(jnp.int32, z.shape, 0) + pl.program_id(0) * BLOCK_B
    out_ref[...] = jnp.where((rows == 0) & (cols == 1), 1.0, out)


def kernel(x, gumbels):
    return pl.pallas_call(
        _fused_body,
        grid=(B // BLOCK_B,),
        in_specs=[
            pl.BlockSpec((BLOCK_B, N), lambda i: (i, 0)),
            pl.BlockSpec((BLOCK_B, N), lambda i: (i, 0)),
        ],
        out_specs=pl.BlockSpec((BLOCK_B, N), lambda i: (i, 0)),
        out_shape=jax.ShapeDtypeStruct((B, N), jnp.float32),
        compiler_params=pltpu.CompilerParams(
            dimension_semantics=("parallel",),
        ),
    )(x, gumbels)
